# baseline (speedup 1.0000x reference)
import functools

import jax
import jax.numpy as jnp
from jax import lax
from jax.experimental import pallas as pl
from jax.experimental.pallas import tpu as pltpu
from jax.experimental.pallas import tpu_sc as plsc

NUM_EMBEDDINGS = 1000000
EMBEDDING_DIM = 64
PAIR_DIM = 2 * EMBEDDING_DIM
TABLE_ELEMS = NUM_EMBEDDINGS * EMBEDDING_DIM
BATCH = 4096 * 200
OUT_ROWS = BATCH // 2

NUM_CORES = 2
NUM_SUBCORES = 16
NUM_WORKERS = NUM_CORES * NUM_SUBCORES
LANES = 16

GROUP = 256
TOK_PER_WORKER = BATCH // NUM_WORKERS
NGRP = TOK_PER_WORKER // GROUP
GROUP_OUT_ROWS = GROUP // 2

_mesh = plsc.VectorSubcoreMesh(core_axis_name="c", subcore_axis_name="s")


@functools.partial(
    pl.kernel,
    out_type=jax.ShapeDtypeStruct((OUT_ROWS, PAIR_DIM), jnp.float32),
    mesh=_mesh,
    scratch_types=[
        pltpu.VMEM((TOK_PER_WORKER,), jnp.int32),
        pltpu.VMEM((2, GROUP * PAIR_DIM), jnp.float32),
        pltpu.VMEM((2, GROUP_OUT_ROWS, PAIR_DIM), jnp.float32),
        pltpu.VMEM_SHARED((NUM_SUBCORES, 2, GROUP), jnp.int32),
        pltpu.SMEM((2, GROUP), jnp.int32),
        pltpu.SemaphoreType.DMA,
        pltpu.SemaphoreType.DMA,
    ],
)
def _embed_sc(
    table_hbm, idx_hbm, out_hbm, idx_v, pair_v, out_v, idx_sh, idx_s, sem_g, sem_s
):
    sid = lax.axis_index("s")
    wid = sid * NUM_CORES + lax.axis_index("c")
    tok_base = wid * TOK_PER_WORKER
    out_base = tok_base // 2
    pltpu.sync_copy(
        idx_hbm.at[pl.ds(pl.multiple_of(tok_base, 128), TOK_PER_WORKER)], idx_v
    )

    def stage_idx(grp, buf):
        pltpu.sync_copy(idx_v.at[pl.ds(grp * GROUP, GROUP)], idx_sh.at[sid, buf])
        pltpu.sync_copy(idx_sh.at[sid, buf], idx_s.at[buf])

    def fire_gathers(buf):
        @pl.loop(0, GROUP, unroll=8)
        def _row(t):
            r = idx_s[buf, t]
            pair_off = lax.shift_right_logical(r, 1) * PAIR_DIM
            pltpu.async_copy(
                table_hbm.at[pl.ds(pl.multiple_of(pair_off, 128), PAIR_DIM)],
                pair_v.at[buf, pl.ds(t * PAIR_DIM, PAIR_DIM)],
                sem_g,
            )

    stage_idx(0, 0)
    fire_gathers(0)

    @pl.loop(0, NGRP)
    def _group(g):
        buf = lax.rem(g, 2)

        pltpu.make_async_copy(
            table_hbm.at[pl.ds(0, GROUP * PAIR_DIM)], pair_v.at[buf], sem_g
        ).wait()

        @pl.when(g + 1 < NGRP)
        def _():
            stage_idx(g + 1, 1 - buf)
            fire_gathers(1 - buf)

        @pl.when(g >= 2)
        def _():
            pltpu.make_async_copy(
                out_v.at[buf], out_hbm.at[pl.ds(0, GROUP_OUT_ROWS)], sem_s
            ).wait()

        @pl.loop(0, GROUP, unroll=8)
        def _extract(t):
            h = lax.rem(idx_s[buf, t], 2) * EMBEDDING_DIM
            row = lax.shift_right_logical(t, 1)
            col = lax.rem(t, 2) * EMBEDDING_DIM
            for k in range(EMBEDDING_DIM // LANES):
                out_v[buf, row, pl.ds(col + k * LANES, LANES)] = pair_v[
                    buf, pl.ds(t * PAIR_DIM + h + k * LANES, LANES)
                ]

        pltpu.async_copy(
            out_v.at[buf],
            out_hbm.at[
                pl.ds(pl.multiple_of(out_base + g * GROUP_OUT_ROWS, 8), GROUP_OUT_ROWS)
            ],
            sem_s,
        )

    pltpu.make_async_copy(
        out_v.at[0], out_hbm.at[pl.ds(0, GROUP_OUT_ROWS)], sem_s
    ).wait()
    pltpu.make_async_copy(
        out_v.at[1], out_hbm.at[pl.ds(0, GROUP_OUT_ROWS)], sem_s
    ).wait()


def kernel(token_ids, weight):
    idx = token_ids.astype(jnp.int32).reshape(BATCH)
    table = weight.reshape(TABLE_ELEMS)
    out = _embed_sc(table, idx)
    return out.reshape(token_ids.shape[0], token_ids.shape[1], EMBEDDING_DIM)
